# baseline (speedup 1.0000x reference)
import jax
import jax.numpy as jnp
from jax import lax
from jax.experimental import pallas as pl
from jax.experimental.pallas import tpu as pltpu
from jax.experimental.pallas import tpu_sc as plsc

_START_WARM = 20000
_END_WARM = 70000
_TOP_P = 0.15

_B, _C, _H, _W = 16, 19, 512, 512
_NPIX = _B * _H * _W
_K = int(_NPIX * _TOP_P)
_NROWS = _NPIX // _W

_ROWS = 64
_NB = 256
_HIST_MAX = 32.0
_INV_W = _NB / _HIST_MAX
_SUB = 8
_GRP = 16 * _SUB
_NW = 32
_TROWS = _NROWS // _NW
_CROWS = 16


def _ce_body(x_ref, t_ref, nll_ref, sum_ref):
    b = pl.program_id(0)
    r = pl.program_id(1)
    x = x_ref[0]
    t = t_ref[0]
    m = jnp.max(x, axis=0)
    e = jnp.exp(x - m[None])
    s = jnp.sum(e, axis=0)
    lse = m + jnp.log(s)
    xt = jnp.zeros_like(m)
    for c in range(_C):
        xt = jnp.where(t == c, x[c], xt)
    nll = lse - xt
    nll_ref[...] = nll

    @pl.when((b == 0) & (r == 0))
    def _():
        sum_ref[0, 0] = 0.0

    sum_ref[0, 0] += jnp.sum(nll)


def _ce_call(output, target):
    rblocks = _H // _ROWS
    grid = (_B, rblocks)
    return pl.pallas_call(
        _ce_body,
        grid=grid,
        in_specs=[
            pl.BlockSpec((1, _C, _ROWS, _W), lambda b, r: (b, 0, r, 0)),
            pl.BlockSpec((1, _ROWS, _W), lambda b, r: (b, r, 0)),
        ],
        out_specs=[
            pl.BlockSpec((_ROWS, _W), lambda b, r: (b * rblocks + r, 0)),
            pl.BlockSpec(memory_space=pltpu.SMEM),
        ],
        out_shape=[
            jax.ShapeDtypeStruct((_NROWS, _W), jnp.float32),
            jax.ShapeDtypeStruct((1, 1), jnp.float32),
        ],
    )(output, target)


def _hist_body(nll_hbm, cnt_out, sum_out, cnt_tab, sum_tab, buf0, buf1,
               sem0, sem1):
    c = lax.axis_index("c")
    s = lax.axis_index("s")
    wid = s * 2 + c
    lanes = lax.iota(jnp.int32, 16)
    ones = jnp.full((16,), 1.0, jnp.float32)
    zeros = jnp.zeros((16,), jnp.float32)
    sublanes = [lanes + su * 16 for su in range(_SUB)]

    def _zero(i, carry):
        for u in range(4):
            cnt_tab[pl.ds((i * 4 + u) * 16, 16)] = zeros
            sum_tab[pl.ds((i * 4 + u) * 16, 16)] = zeros
        return carry

    lax.fori_loop(0, (_GRP * _NB) // (16 * 4), _zero, 0)

    base_row = wid * _TROWS
    npairs = _TROWS // (2 * _CROWS)

    def _rows(buf, r, carry):
        for j in range(_W // 16):
            v = buf[r, pl.ds(j * 16, 16)]
            b = jnp.clip(v * _INV_W, 0.0, float(_NB - 1))
            idx = b.astype(jnp.int32) * _GRP + sublanes[j % _SUB]
            plsc.addupdate_scatter(cnt_tab, [idx], ones)
            plsc.addupdate_scatter(sum_tab, [idx], v)
        return carry

    def _start(g, buf, sem):
        return pltpu.async_copy(
            nll_hbm.at[pl.ds(base_row + g * _CROWS, _CROWS)], buf, sem
        )

    def _wait(g, buf, sem):
        pltpu.make_async_copy(
            nll_hbm.at[pl.ds(base_row + g * _CROWS, _CROWS)], buf, sem
        ).wait()

    _start(0, buf0, sem0)

    def _pair(h, carry):
        g0 = h * 2
        _start(g0 + 1, buf1, sem1)
        _wait(g0, buf0, sem0)
        lax.fori_loop(0, _CROWS, lambda r, cc: _rows(buf0, r, cc), carry)

        @pl.when(h < npairs - 1)
        def _():
            _start(g0 + 2, buf0, sem0)

        _wait(g0 + 1, buf1, sem1)
        lax.fori_loop(0, _CROWS, lambda r, cc: _rows(buf1, r, cc), carry)
        return carry

    lax.fori_loop(0, npairs, _pair, 0)

    pltpu.sync_copy(cnt_tab, cnt_out.at[wid])
    pltpu.sync_copy(sum_tab, sum_out.at[wid])


def _hist_call(nll):
    mesh = plsc.VectorSubcoreMesh(core_axis_name="c", subcore_axis_name="s")
    fn = pl.kernel(
        _hist_body,
        out_type=(
            jax.ShapeDtypeStruct((_NW, _GRP * _NB), jnp.float32),
            jax.ShapeDtypeStruct((_NW, _GRP * _NB), jnp.float32),
        ),
        mesh=mesh,
        compiler_params=pltpu.CompilerParams(needs_layout_passes=False),
        scratch_types=[
            pltpu.VMEM((_GRP * _NB,), jnp.float32),
            pltpu.VMEM((_GRP * _NB,), jnp.float32),
            pltpu.VMEM((_CROWS, _W), jnp.float32),
            pltpu.VMEM((_CROWS, _W), jnp.float32),
            pltpu.SemaphoreType.DMA,
            pltpu.SemaphoreType.DMA,
        ],
    )
    return fn(nll)


_NT = _GRP * _NB


def _suffix_incl(x):
    n = x.shape[1]
    sft = 1
    while sft < n:
        x = x + jnp.concatenate(
            [x[:, sft:], jnp.zeros((1, sft), jnp.float32)], axis=1
        )
        sft *= 2
    return x


def _group_suffix(x, grp):
    sft = 1
    while sft < _GRP:
        sh = jnp.concatenate(
            [x[:, sft:], jnp.zeros((1, sft), jnp.float32)], axis=1
        )
        x = x + jnp.where(grp < _GRP - sft, sh, 0.0)
        sft *= 2
    return x


def _sel_body(cnt_ref, sum_ref, tot_ref, topk_ref, raw_ref):
    cnt = jnp.sum(cnt_ref[...], axis=0, keepdims=True)
    sm = jnp.sum(sum_ref[...], axis=0, keepdims=True)
    pos = lax.broadcasted_iota(jnp.int32, (1, _NT), 1)
    grp = pos % _GRP
    base = (grp == 0).astype(jnp.float32)
    counts = _group_suffix(cnt, grp) * base
    sums = _group_suffix(sm, grp) * base
    rc = _suffix_incl(counts)
    rs = _suffix_incl(sums)
    above_c = rc - counts
    above_s = rs - sums
    kf = jnp.float32(_K)
    hit = ((above_c < kf) & (above_c + counts >= kf)).astype(jnp.float32)
    hit = hit * base
    center = ((pos // _GRP).astype(jnp.float32) + 0.5) * (1.0 / _INV_W)
    a_sel = jnp.sum(above_c * hit)
    s_sel = jnp.sum(above_s * hit)
    t_sel = jnp.sum(center * hit)
    topk_sum = s_sel + (kf - a_sel) * t_sel
    topk_ref[0, 0] = topk_sum / kf
    raw_ref[0, 0] = tot_ref[0, 0] / jnp.float32(_NPIX)


def _sel_call(cnt, sm, tot):
    return pl.pallas_call(
        _sel_body,
        in_specs=[
            pl.BlockSpec(memory_space=pltpu.VMEM),
            pl.BlockSpec(memory_space=pltpu.VMEM),
            pl.BlockSpec(memory_space=pltpu.SMEM),
        ],
        out_specs=[
            pl.BlockSpec(memory_space=pltpu.SMEM),
            pl.BlockSpec(memory_space=pltpu.SMEM),
        ],
        out_shape=[
            jax.ShapeDtypeStruct((1, 1), jnp.float32),
            jax.ShapeDtypeStruct((1, 1), jnp.float32),
        ],
    )(cnt, sm, tot)


def kernel(output, target, it):
    nll, tot = _ce_call(output, target)
    cnt, sm = _hist_call(nll)
    topk, raw = _sel_call(cnt, sm, tot)
    topk_mean = topk[0, 0]
    raw_mean = raw[0, 0]

    it_arr = jnp.asarray(it)
    itf = it_arr.astype(jnp.float32)
    ramp = jnp.float32(_TOP_P) + jnp.float32(1.0 - _TOP_P) * (
        (jnp.float32(_END_WARM) - itf) / jnp.float32(_END_WARM - _START_WARM)
    )
    this_p = jnp.where(
        it_arr < _START_WARM,
        jnp.float32(1.0),
        jnp.where(it_arr > _END_WARM, jnp.float32(_TOP_P), ramp),
    )
    loss = jnp.where(it_arr < _START_WARM, raw_mean, topk_mean)
    return (loss, this_p, raw_mean)


# speedup vs baseline: 1.0259x; 1.0259x over previous
import jax
import jax.numpy as jnp
from jax import lax
from jax.experimental import pallas as pl
from jax.experimental.pallas import tpu as pltpu
from jax.experimental.pallas import tpu_sc as plsc

_START_WARM = 20000
_END_WARM = 70000
_TOP_P = 0.15

_B, _C, _H, _W = 16, 19, 512, 512
_NPIX = _B * _H * _W
_K = int(_NPIX * _TOP_P)
_NROWS = _NPIX // _W

_ROWS = 64
_NB = 256
_HIST_MAX = 32.0
_INV_W = _NB / _HIST_MAX
_SUB = 8
_GRP = 16 * _SUB
_NW = 32
_TROWS = _NROWS // _NW
_CROWS = 16


def _ce_body(x_ref, t_ref, nll_ref, sum_ref):
    b = pl.program_id(0)
    r = pl.program_id(1)
    x = x_ref[0]
    t = t_ref[0]
    m = jnp.max(x, axis=0)
    e = jnp.exp(x - m[None])
    s = jnp.sum(e, axis=0)
    lse = m + jnp.log(s)
    xt = jnp.zeros_like(m)
    for c in range(_C):
        xt = jnp.where(t == c, x[c], xt)
    nll = lse - xt
    nll_ref[...] = nll

    @pl.when((b == 0) & (r == 0))
    def _():
        sum_ref[0, 0] = 0.0

    sum_ref[0, 0] += jnp.sum(nll)


def _ce_call(output, target):
    rblocks = _H // _ROWS
    grid = (_B, rblocks)
    return pl.pallas_call(
        _ce_body,
        grid=grid,
        in_specs=[
            pl.BlockSpec((1, _C, _ROWS, _W), lambda b, r: (b, 0, r, 0)),
            pl.BlockSpec((1, _ROWS, _W), lambda b, r: (b, r, 0)),
        ],
        out_specs=[
            pl.BlockSpec((_ROWS, _W), lambda b, r: (b * rblocks + r, 0)),
            pl.BlockSpec(memory_space=pltpu.SMEM),
        ],
        out_shape=[
            jax.ShapeDtypeStruct((_NROWS, _W), jnp.float32),
            jax.ShapeDtypeStruct((1, 1), jnp.float32),
        ],
    )(output, target)


def _hist_body(nll_hbm, cnt_out, sum_out, cnt_tab, sum_tab, buf0, buf1,
               sem0, sem1):
    c = lax.axis_index("c")
    s = lax.axis_index("s")
    wid = s * 2 + c
    lanes = lax.iota(jnp.int32, 16)
    ones = jnp.full((16,), 1.0, jnp.float32)
    zeros = jnp.zeros((16,), jnp.float32)
    sublanes = [lanes + su * (_NB * 16) for su in range(_SUB)]

    def _zero(i, carry):
        for u in range(4):
            cnt_tab[pl.ds((i * 4 + u) * 16, 16)] = zeros
            sum_tab[pl.ds((i * 4 + u) * 16, 16)] = zeros
        return carry

    lax.fori_loop(0, (_GRP * _NB) // (16 * 4), _zero, 0)

    base_row = wid * _TROWS
    npairs = _TROWS // (2 * _CROWS)

    def _rows(buf, r, carry):
        for j in range(_W // 16):
            v = buf[r, pl.ds(j * 16, 16)]
            b = jnp.clip(v * _INV_W, 0.0, float(_NB - 1))
            idx = b.astype(jnp.int32) * 16 + sublanes[j % _SUB]
            plsc.addupdate_scatter(cnt_tab, [idx], ones)
            plsc.addupdate_scatter(sum_tab, [idx], v)
        return carry

    def _start(g, buf, sem):
        return pltpu.async_copy(
            nll_hbm.at[pl.ds(base_row + g * _CROWS, _CROWS)], buf, sem
        )

    def _wait(g, buf, sem):
        pltpu.make_async_copy(
            nll_hbm.at[pl.ds(base_row + g * _CROWS, _CROWS)], buf, sem
        ).wait()

    _start(0, buf0, sem0)

    def _pair(h, carry):
        g0 = h * 2
        _start(g0 + 1, buf1, sem1)
        _wait(g0, buf0, sem0)
        lax.fori_loop(0, _CROWS, lambda r, cc: _rows(buf0, r, cc), carry)

        @pl.when(h < npairs - 1)
        def _():
            _start(g0 + 2, buf0, sem0)

        _wait(g0 + 1, buf1, sem1)
        lax.fori_loop(0, _CROWS, lambda r, cc: _rows(buf1, r, cc), carry)
        return carry

    lax.fori_loop(0, npairs, _pair, 0)

    pltpu.sync_copy(cnt_tab, cnt_out.at[wid])
    pltpu.sync_copy(sum_tab, sum_out.at[wid])


def _hist_call(nll):
    mesh = plsc.VectorSubcoreMesh(core_axis_name="c", subcore_axis_name="s")
    fn = pl.kernel(
        _hist_body,
        out_type=(
            jax.ShapeDtypeStruct((_NW, _GRP * _NB), jnp.float32),
            jax.ShapeDtypeStruct((_NW, _GRP * _NB), jnp.float32),
        ),
        mesh=mesh,
        compiler_params=pltpu.CompilerParams(needs_layout_passes=False),
        scratch_types=[
            pltpu.VMEM((_GRP * _NB,), jnp.float32),
            pltpu.VMEM((_GRP * _NB,), jnp.float32),
            pltpu.VMEM((_CROWS, _W), jnp.float32),
            pltpu.VMEM((_CROWS, _W), jnp.float32),
            pltpu.SemaphoreType.DMA,
            pltpu.SemaphoreType.DMA,
        ],
    )
    return fn(nll)


_NT = _GRP * _NB


def _suffix_incl(x):
    n = x.shape[1]
    sft = 1
    while sft < n:
        x = x + jnp.concatenate(
            [x[:, sft:], jnp.zeros((1, sft), jnp.float32)], axis=1
        )
        sft *= 2
    return x


def _group_suffix(x, grp, width):
    sft = 1
    while sft < width:
        sh = jnp.concatenate(
            [x[:, sft:], jnp.zeros((1, sft), jnp.float32)], axis=1
        )
        x = x + jnp.where(grp < width - sft, sh, 0.0)
        sft *= 2
    return x


def _sel_body(cnt_ref, sum_ref, tot_ref, topk_ref, raw_ref):
    cnt_full = jnp.sum(cnt_ref[...], axis=0, keepdims=True)
    sm_full = jnp.sum(sum_ref[...], axis=0, keepdims=True)
    nsub = _NB * 16
    cnt = jnp.zeros((1, nsub), jnp.float32)
    sm = jnp.zeros((1, nsub), jnp.float32)
    for su in range(_SUB):
        sl = slice(su * nsub, (su + 1) * nsub)
        cnt = cnt + cnt_full[:, sl]
        sm = sm + sm_full[:, sl]
    pos = lax.broadcasted_iota(jnp.int32, (1, nsub), 1)
    grp = pos % 16
    base = (grp == 0).astype(jnp.float32)
    counts = _group_suffix(cnt, grp, 16) * base
    sums = _group_suffix(sm, grp, 16) * base
    rc = _suffix_incl(counts)
    rs = _suffix_incl(sums)
    above_c = rc - counts
    above_s = rs - sums
    kf = jnp.float32(_K)
    hit = ((above_c < kf) & (above_c + counts >= kf)).astype(jnp.float32)
    hit = hit * base
    center = ((pos // 16).astype(jnp.float32) + 0.5) * (1.0 / _INV_W)
    a_sel = jnp.sum(above_c * hit)
    s_sel = jnp.sum(above_s * hit)
    t_sel = jnp.sum(center * hit)
    topk_sum = s_sel + (kf - a_sel) * t_sel
    topk_ref[0, 0] = topk_sum / kf
    raw_ref[0, 0] = tot_ref[0, 0] / jnp.float32(_NPIX)


def _sel_call(cnt, sm, tot):
    return pl.pallas_call(
        _sel_body,
        in_specs=[
            pl.BlockSpec(memory_space=pltpu.VMEM),
            pl.BlockSpec(memory_space=pltpu.VMEM),
            pl.BlockSpec(memory_space=pltpu.SMEM),
        ],
        out_specs=[
            pl.BlockSpec(memory_space=pltpu.SMEM),
            pl.BlockSpec(memory_space=pltpu.SMEM),
        ],
        out_shape=[
            jax.ShapeDtypeStruct((1, 1), jnp.float32),
            jax.ShapeDtypeStruct((1, 1), jnp.float32),
        ],
    )(cnt, sm, tot)


def kernel(output, target, it):
    nll, tot = _ce_call(output, target)
    cnt, sm = _hist_call(nll)
    topk, raw = _sel_call(cnt, sm, tot)
    topk_mean = topk[0, 0]
    raw_mean = raw[0, 0]

    it_arr = jnp.asarray(it)
    itf = it_arr.astype(jnp.float32)
    ramp = jnp.float32(_TOP_P) + jnp.float32(1.0 - _TOP_P) * (
        (jnp.float32(_END_WARM) - itf) / jnp.float32(_END_WARM - _START_WARM)
    )
    this_p = jnp.where(
        it_arr < _START_WARM,
        jnp.float32(1.0),
        jnp.where(it_arr > _END_WARM, jnp.float32(_TOP_P), ramp),
    )
    loss = jnp.where(it_arr < _START_WARM, raw_mean, topk_mean)
    return (loss, this_p, raw_mean)


# speedup vs baseline: 1.2984x; 1.2656x over previous
import jax
import jax.numpy as jnp
from jax import lax
from jax.experimental import pallas as pl
from jax.experimental.pallas import tpu as pltpu
from jax.experimental.pallas import tpu_sc as plsc

_START_WARM = 20000
_END_WARM = 70000
_TOP_P = 0.15

_B, _C, _H, _W = 16, 19, 512, 512
_NPIX = _B * _H * _W
_K = int(_NPIX * _TOP_P)
_NQ = 4
_BQ = _B // _NQ
_QROWS = _BQ * _H

_ROWS = 64
_NB = 256
_HIST_MAX = 32.0
_INV_W = _NB / _HIST_MAX
_SUB = 8
_NSB = _NB * 16
_NW = 32
_TROWS = _QROWS // _NW
_CROWS = 16


def _ce_body(x_ref, t_ref, nll_ref, sum_ref):
    b = pl.program_id(0)
    r = pl.program_id(1)
    x = x_ref[0]
    t = t_ref[0]
    m = jnp.max(x, axis=0)
    e = jnp.exp(x - m[None])
    s = jnp.sum(e, axis=0)
    lse = m + jnp.log(s)
    xt = jnp.zeros_like(m)
    for c in range(_C):
        xt = jnp.where(t == c, x[c], xt)
    nll = lse - xt
    nll_ref[...] = nll

    @pl.when((b == 0) & (r == 0))
    def _():
        sum_ref[0, 0] = 0.0

    sum_ref[0, 0] += jnp.sum(nll)


def _ce_call(output, target, q):
    rblocks = _H // _ROWS
    grid = (_BQ, rblocks)
    return pl.pallas_call(
        _ce_body,
        grid=grid,
        in_specs=[
            pl.BlockSpec(
                (1, _C, _ROWS, _W), lambda b, r: (q * _BQ + b, 0, r, 0)
            ),
            pl.BlockSpec((1, _ROWS, _W), lambda b, r: (q * _BQ + b, r, 0)),
        ],
        out_specs=[
            pl.BlockSpec((_ROWS, _W), lambda b, r: (b * rblocks + r, 0)),
            pl.BlockSpec(memory_space=pltpu.SMEM),
        ],
        out_shape=[
            jax.ShapeDtypeStruct((_QROWS, _W), jnp.float32),
            jax.ShapeDtypeStruct((1, 1), jnp.float32),
        ],
    )(output, target)


def _hist_body(nll_hbm, cnt_out, sum_out, cnt_tab, sum_tab, cnt_m, sum_m,
               buf0, buf1, sem0, sem1):
    c = lax.axis_index("c")
    s = lax.axis_index("s")
    wid = s * 2 + c
    lanes = lax.iota(jnp.int32, 16)
    ones = jnp.full((16,), 1.0, jnp.float32)
    zeros = jnp.zeros((16,), jnp.float32)
    sublanes = [lanes + su * _NSB for su in range(_SUB)]

    def _zero(i, carry):
        for u in range(4):
            cnt_tab[pl.ds((i * 4 + u) * 16, 16)] = zeros
            sum_tab[pl.ds((i * 4 + u) * 16, 16)] = zeros
        return carry

    lax.fori_loop(0, (_SUB * _NSB) // (16 * 4), _zero, 0)

    base_row = wid * _TROWS
    npairs = _TROWS // (2 * _CROWS)

    def _rows(buf, r, carry):
        for j in range(_W // 16):
            v = buf[r, pl.ds(j * 16, 16)]
            b = jnp.clip(v * _INV_W, 0.0, float(_NB - 1))
            idx = b.astype(jnp.int32) * 16 + sublanes[j % _SUB]
            plsc.addupdate_scatter(cnt_tab, [idx], ones)
            plsc.addupdate_scatter(sum_tab, [idx], v)
        return carry

    def _start(g, buf, sem):
        return pltpu.async_copy(
            nll_hbm.at[pl.ds(base_row + g * _CROWS, _CROWS)], buf, sem
        )

    def _wait(g, buf, sem):
        pltpu.make_async_copy(
            nll_hbm.at[pl.ds(base_row + g * _CROWS, _CROWS)], buf, sem
        ).wait()

    _start(0, buf0, sem0)

    def _pair(h, carry):
        g0 = h * 2
        _start(g0 + 1, buf1, sem1)
        _wait(g0, buf0, sem0)
        lax.fori_loop(0, _CROWS, lambda r, cc: _rows(buf0, r, cc), carry)

        @pl.when(h < npairs - 1)
        def _():
            _start(g0 + 2, buf0, sem0)

        _wait(g0 + 1, buf1, sem1)
        lax.fori_loop(0, _CROWS, lambda r, cc: _rows(buf1, r, cc), carry)
        return carry

    lax.fori_loop(0, npairs, _pair, 0)

    def _fold(w, carry):
        ca = cnt_tab[pl.ds(w * 16, 16)]
        sa = sum_tab[pl.ds(w * 16, 16)]
        for su in range(1, _SUB):
            ca = ca + cnt_tab[pl.ds(su * _NSB + w * 16, 16)]
            sa = sa + sum_tab[pl.ds(su * _NSB + w * 16, 16)]
        cnt_m[pl.ds(w * 16, 16)] = ca
        sum_m[pl.ds(w * 16, 16)] = sa
        return carry

    lax.fori_loop(0, _NSB // 16, _fold, 0)

    pltpu.sync_copy(cnt_m, cnt_out.at[wid])
    pltpu.sync_copy(sum_m, sum_out.at[wid])


def _hist_call(nll):
    mesh = plsc.VectorSubcoreMesh(core_axis_name="c", subcore_axis_name="s")
    fn = pl.kernel(
        _hist_body,
        out_type=(
            jax.ShapeDtypeStruct((_NW, _NSB), jnp.float32),
            jax.ShapeDtypeStruct((_NW, _NSB), jnp.float32),
        ),
        mesh=mesh,
        compiler_params=pltpu.CompilerParams(needs_layout_passes=False),
        scratch_types=[
            pltpu.VMEM((_SUB * _NSB,), jnp.float32),
            pltpu.VMEM((_SUB * _NSB,), jnp.float32),
            pltpu.VMEM((_NSB,), jnp.float32),
            pltpu.VMEM((_NSB,), jnp.float32),
            pltpu.VMEM((_CROWS, _W), jnp.float32),
            pltpu.VMEM((_CROWS, _W), jnp.float32),
            pltpu.SemaphoreType.DMA,
            pltpu.SemaphoreType.DMA,
        ],
    )
    return fn(nll)


def _suffix_incl(x):
    n = x.shape[1]
    sft = 1
    while sft < n:
        x = x + jnp.concatenate(
            [x[:, sft:], jnp.zeros((1, sft), jnp.float32)], axis=1
        )
        sft *= 2
    return x


def _group_suffix(x, grp, width):
    sft = 1
    while sft < width:
        sh = jnp.concatenate(
            [x[:, sft:], jnp.zeros((1, sft), jnp.float32)], axis=1
        )
        x = x + jnp.where(grp < width - sft, sh, 0.0)
        sft *= 2
    return x


def _sel_body(*refs):
    cnt_refs = refs[:_NQ]
    sum_refs = refs[_NQ : 2 * _NQ]
    tot_ref = refs[2 * _NQ]
    topk_ref, raw_ref = refs[2 * _NQ + 1], refs[2 * _NQ + 2]
    cnt = jnp.zeros((1, _NSB), jnp.float32)
    sm = jnp.zeros((1, _NSB), jnp.float32)
    for q in range(_NQ):
        cnt = cnt + jnp.sum(cnt_refs[q][...], axis=0, keepdims=True)
        sm = sm + jnp.sum(sum_refs[q][...], axis=0, keepdims=True)
    pos = lax.broadcasted_iota(jnp.int32, (1, _NSB), 1)
    grp = pos % 16
    base = (grp == 0).astype(jnp.float32)
    counts = _group_suffix(cnt, grp, 16) * base
    sums = _group_suffix(sm, grp, 16) * base
    rc = _suffix_incl(counts)
    rs = _suffix_incl(sums)
    above_c = rc - counts
    above_s = rs - sums
    kf = jnp.float32(_K)
    hit = ((above_c < kf) & (above_c + counts >= kf)).astype(jnp.float32)
    hit = hit * base
    center = ((pos // 16).astype(jnp.float32) + 0.5) * (1.0 / _INV_W)
    a_sel = jnp.sum(above_c * hit)
    s_sel = jnp.sum(above_s * hit)
    t_sel = jnp.sum(center * hit)
    topk_sum = s_sel + (kf - a_sel) * t_sel
    topk_ref[0, 0] = topk_sum / kf
    raw_ref[0, 0] = tot_ref[0, 0] / jnp.float32(_NPIX)


def _sel_call(cnts, sms, tot):
    return pl.pallas_call(
        _sel_body,
        in_specs=(
            [pl.BlockSpec(memory_space=pltpu.VMEM) for _ in range(2 * _NQ)]
            + [pl.BlockSpec(memory_space=pltpu.SMEM)]
        ),
        out_specs=[
            pl.BlockSpec(memory_space=pltpu.SMEM),
            pl.BlockSpec(memory_space=pltpu.SMEM),
        ],
        out_shape=[
            jax.ShapeDtypeStruct((1, 1), jnp.float32),
            jax.ShapeDtypeStruct((1, 1), jnp.float32),
        ],
    )(*cnts, *sms, tot)


def kernel(output, target, it):
    cnts, sms, tots = [], [], []
    for q in range(_NQ):
        nll_q, tot_q = _ce_call(output, target, q)
        cnt_q, sm_q = _hist_call(nll_q)
        cnts.append(cnt_q)
        sms.append(sm_q)
        tots.append(tot_q)
    tot = tots[0] + tots[1] + tots[2] + tots[3]
    topk, raw = _sel_call(cnts, sms, tot)
    topk_mean = topk[0, 0]
    raw_mean = raw[0, 0]

    it_arr = jnp.asarray(it)
    itf = it_arr.astype(jnp.float32)
    ramp = jnp.float32(_TOP_P) + jnp.float32(1.0 - _TOP_P) * (
        (jnp.float32(_END_WARM) - itf) / jnp.float32(_END_WARM - _START_WARM)
    )
    this_p = jnp.where(
        it_arr < _START_WARM,
        jnp.float32(1.0),
        jnp.where(it_arr > _END_WARM, jnp.float32(_TOP_P), ramp),
    )
    loss = jnp.where(it_arr < _START_WARM, raw_mean, topk_mean)
    return (loss, this_p, raw_mean)


# speedup vs baseline: 1.3031x; 1.0036x over previous
import jax
import jax.numpy as jnp
from jax import lax
from jax.experimental import pallas as pl
from jax.experimental.pallas import tpu as pltpu
from jax.experimental.pallas import tpu_sc as plsc

_START_WARM = 20000
_END_WARM = 70000
_TOP_P = 0.15

_B, _C, _H, _W = 16, 19, 512, 512
_NPIX = _B * _H * _W
_K = int(_NPIX * _TOP_P)
_NQ = 8
_BQ = _B // _NQ
_QROWS = _BQ * _H

_ROWS = 64
_NB = 256
_HIST_MAX = 32.0
_INV_W = _NB / _HIST_MAX
_SUB = 8
_NSB = _NB * 16
_NW = 32
_TROWS = _QROWS // _NW
_CROWS = 16


def _ce_body(x_ref, t_ref, nll_ref, sum_ref):
    b = pl.program_id(0)
    r = pl.program_id(1)
    x = x_ref[0]
    t = t_ref[0]
    m = jnp.max(x, axis=0)
    e = jnp.exp(x - m[None])
    s = jnp.sum(e, axis=0)
    lse = m + jnp.log(s)
    xt = jnp.zeros_like(m)
    for c in range(_C):
        xt = jnp.where(t == c, x[c], xt)
    nll = lse - xt
    nll_ref[...] = nll

    @pl.when((b == 0) & (r == 0))
    def _():
        sum_ref[0, 0] = 0.0

    sum_ref[0, 0] += jnp.sum(nll)


def _ce_call(output, target, q):
    rblocks = _H // _ROWS
    grid = (_BQ, rblocks)
    return pl.pallas_call(
        _ce_body,
        grid=grid,
        in_specs=[
            pl.BlockSpec(
                (1, _C, _ROWS, _W), lambda b, r: (q * _BQ + b, 0, r, 0)
            ),
            pl.BlockSpec((1, _ROWS, _W), lambda b, r: (q * _BQ + b, r, 0)),
        ],
        out_specs=[
            pl.BlockSpec((_ROWS, _W), lambda b, r: (b * rblocks + r, 0)),
            pl.BlockSpec(memory_space=pltpu.SMEM),
        ],
        out_shape=[
            jax.ShapeDtypeStruct((_QROWS, _W), jnp.float32),
            jax.ShapeDtypeStruct((1, 1), jnp.float32),
        ],
    )(output, target)


def _hist_body(nll_hbm, cnt_out, sum_out, cnt_tab, sum_tab, cnt_m, sum_m,
               buf0, buf1, sem0, sem1):
    c = lax.axis_index("c")
    s = lax.axis_index("s")
    wid = s * 2 + c
    lanes = lax.iota(jnp.int32, 16)
    ones = jnp.full((16,), 1.0, jnp.float32)
    zeros = jnp.zeros((16,), jnp.float32)
    sublanes = [lanes + su * _NSB for su in range(_SUB)]

    def _zero(i, carry):
        for u in range(4):
            cnt_tab[pl.ds((i * 4 + u) * 16, 16)] = zeros
            sum_tab[pl.ds((i * 4 + u) * 16, 16)] = zeros
        return carry

    lax.fori_loop(0, (_SUB * _NSB) // (16 * 4), _zero, 0)

    base_row = wid * _TROWS
    npairs = _TROWS // (2 * _CROWS)

    def _rows(buf, r, carry):
        for j in range(_W // 16):
            v = buf[r, pl.ds(j * 16, 16)]
            b = jnp.clip(v * _INV_W, 0.0, float(_NB - 1))
            idx = b.astype(jnp.int32) * 16 + sublanes[j % _SUB]
            plsc.addupdate_scatter(cnt_tab, [idx], ones)
            plsc.addupdate_scatter(sum_tab, [idx], v)
        return carry

    def _start(g, buf, sem):
        return pltpu.async_copy(
            nll_hbm.at[pl.ds(base_row + g * _CROWS, _CROWS)], buf, sem
        )

    def _wait(g, buf, sem):
        pltpu.make_async_copy(
            nll_hbm.at[pl.ds(base_row + g * _CROWS, _CROWS)], buf, sem
        ).wait()

    _start(0, buf0, sem0)

    def _pair(h, carry):
        g0 = h * 2
        _start(g0 + 1, buf1, sem1)
        _wait(g0, buf0, sem0)
        lax.fori_loop(0, _CROWS, lambda r, cc: _rows(buf0, r, cc), carry)

        @pl.when(h < npairs - 1)
        def _():
            _start(g0 + 2, buf0, sem0)

        _wait(g0 + 1, buf1, sem1)
        lax.fori_loop(0, _CROWS, lambda r, cc: _rows(buf1, r, cc), carry)
        return carry

    lax.fori_loop(0, npairs, _pair, 0)

    def _fold(w, carry):
        ca = cnt_tab[pl.ds(w * 16, 16)]
        sa = sum_tab[pl.ds(w * 16, 16)]
        for su in range(1, _SUB):
            ca = ca + cnt_tab[pl.ds(su * _NSB + w * 16, 16)]
            sa = sa + sum_tab[pl.ds(su * _NSB + w * 16, 16)]
        cnt_m[pl.ds(w * 16, 16)] = ca
        sum_m[pl.ds(w * 16, 16)] = sa
        return carry

    lax.fori_loop(0, _NSB // 16, _fold, 0)

    pltpu.sync_copy(cnt_m, cnt_out.at[wid])
    pltpu.sync_copy(sum_m, sum_out.at[wid])


def _hist_call(nll):
    mesh = plsc.VectorSubcoreMesh(core_axis_name="c", subcore_axis_name="s")
    fn = pl.kernel(
        _hist_body,
        out_type=(
            jax.ShapeDtypeStruct((_NW, _NSB), jnp.float32),
            jax.ShapeDtypeStruct((_NW, _NSB), jnp.float32),
        ),
        mesh=mesh,
        compiler_params=pltpu.CompilerParams(needs_layout_passes=False),
        scratch_types=[
            pltpu.VMEM((_SUB * _NSB,), jnp.float32),
            pltpu.VMEM((_SUB * _NSB,), jnp.float32),
            pltpu.VMEM((_NSB,), jnp.float32),
            pltpu.VMEM((_NSB,), jnp.float32),
            pltpu.VMEM((_CROWS, _W), jnp.float32),
            pltpu.VMEM((_CROWS, _W), jnp.float32),
            pltpu.SemaphoreType.DMA,
            pltpu.SemaphoreType.DMA,
        ],
    )
    return fn(nll)


def _suffix_incl(x):
    n = x.shape[1]
    sft = 1
    while sft < n:
        x = x + jnp.concatenate(
            [x[:, sft:], jnp.zeros((1, sft), jnp.float32)], axis=1
        )
        sft *= 2
    return x


def _group_suffix(x, grp, width):
    sft = 1
    while sft < width:
        sh = jnp.concatenate(
            [x[:, sft:], jnp.zeros((1, sft), jnp.float32)], axis=1
        )
        x = x + jnp.where(grp < width - sft, sh, 0.0)
        sft *= 2
    return x


def _sel_body(*refs):
    cnt_refs = refs[:_NQ]
    sum_refs = refs[_NQ : 2 * _NQ]
    tot_ref = refs[2 * _NQ]
    topk_ref, raw_ref = refs[2 * _NQ + 1], refs[2 * _NQ + 2]
    cnt = jnp.zeros((1, _NSB), jnp.float32)
    sm = jnp.zeros((1, _NSB), jnp.float32)
    for q in range(_NQ):
        cnt = cnt + jnp.sum(cnt_refs[q][...], axis=0, keepdims=True)
        sm = sm + jnp.sum(sum_refs[q][...], axis=0, keepdims=True)
    pos = lax.broadcasted_iota(jnp.int32, (1, _NSB), 1)
    grp = pos % 16
    base = (grp == 0).astype(jnp.float32)
    counts = _group_suffix(cnt, grp, 16) * base
    sums = _group_suffix(sm, grp, 16) * base
    rc = _suffix_incl(counts)
    rs = _suffix_incl(sums)
    above_c = rc - counts
    above_s = rs - sums
    kf = jnp.float32(_K)
    hit = ((above_c < kf) & (above_c + counts >= kf)).astype(jnp.float32)
    hit = hit * base
    center = ((pos // 16).astype(jnp.float32) + 0.5) * (1.0 / _INV_W)
    a_sel = jnp.sum(above_c * hit)
    s_sel = jnp.sum(above_s * hit)
    t_sel = jnp.sum(center * hit)
    topk_sum = s_sel + (kf - a_sel) * t_sel
    topk_ref[0, 0] = topk_sum / kf
    raw_ref[0, 0] = tot_ref[0, 0] / jnp.float32(_NPIX)


def _sel_call(cnts, sms, tot):
    return pl.pallas_call(
        _sel_body,
        in_specs=(
            [pl.BlockSpec(memory_space=pltpu.VMEM) for _ in range(2 * _NQ)]
            + [pl.BlockSpec(memory_space=pltpu.SMEM)]
        ),
        out_specs=[
            pl.BlockSpec(memory_space=pltpu.SMEM),
            pl.BlockSpec(memory_space=pltpu.SMEM),
        ],
        out_shape=[
            jax.ShapeDtypeStruct((1, 1), jnp.float32),
            jax.ShapeDtypeStruct((1, 1), jnp.float32),
        ],
    )(*cnts, *sms, tot)


def kernel(output, target, it):
    cnts, sms, tots = [], [], []
    for q in range(_NQ):
        nll_q, tot_q = _ce_call(output, target, q)
        cnt_q, sm_q = _hist_call(nll_q)
        cnts.append(cnt_q)
        sms.append(sm_q)
        tots.append(tot_q)
    tot = sum(tots[1:], tots[0])
    topk, raw = _sel_call(cnts, sms, tot)
    topk_mean = topk[0, 0]
    raw_mean = raw[0, 0]

    it_arr = jnp.asarray(it)
    itf = it_arr.astype(jnp.float32)
    ramp = jnp.float32(_TOP_P) + jnp.float32(1.0 - _TOP_P) * (
        (jnp.float32(_END_WARM) - itf) / jnp.float32(_END_WARM - _START_WARM)
    )
    this_p = jnp.where(
        it_arr < _START_WARM,
        jnp.float32(1.0),
        jnp.where(it_arr > _END_WARM, jnp.float32(_TOP_P), ramp),
    )
    loss = jnp.where(it_arr < _START_WARM, raw_mean, topk_mean)
    return (loss, this_p, raw_mean)


# speedup vs baseline: 1.3201x; 1.0130x over previous
import jax
import jax.numpy as jnp
from jax import lax
from jax.experimental import pallas as pl
from jax.experimental.pallas import tpu as pltpu
from jax.experimental.pallas import tpu_sc as plsc

_START_WARM = 20000
_END_WARM = 70000
_TOP_P = 0.15

_B, _C, _H, _W = 16, 19, 512, 512
_NPIX = _B * _H * _W
_K = int(_NPIX * _TOP_P)
_NQ = 8
_BQ = _B // _NQ
_QROWS = _BQ * _H

_ROWS = 64
_NB = 256
_HIST_MAX = 32.0
_INV_W = _NB / _HIST_MAX
_SUB = 1
_NSB = _NB * 16
_NW = 32
_TROWS = _QROWS // _NW
_CROWS = 16


def _ce_body(x_ref, t_ref, nll_ref, sum_ref):
    b = pl.program_id(0)
    r = pl.program_id(1)
    x = x_ref[0]
    t = t_ref[0]
    m = jnp.max(x, axis=0)
    e = jnp.exp(x - m[None])
    s = jnp.sum(e, axis=0)
    lse = m + jnp.log(s)
    xt = jnp.zeros_like(m)
    for c in range(_C):
        xt = jnp.where(t == c, x[c], xt)
    nll = lse - xt
    nll_ref[...] = nll

    @pl.when((b == 0) & (r == 0))
    def _():
        sum_ref[0, 0] = 0.0

    sum_ref[0, 0] += jnp.sum(nll)


def _ce_call(output, target, q):
    rblocks = _H // _ROWS
    grid = (_BQ, rblocks)
    return pl.pallas_call(
        _ce_body,
        grid=grid,
        in_specs=[
            pl.BlockSpec(
                (1, _C, _ROWS, _W), lambda b, r: (q * _BQ + b, 0, r, 0)
            ),
            pl.BlockSpec((1, _ROWS, _W), lambda b, r: (q * _BQ + b, r, 0)),
        ],
        out_specs=[
            pl.BlockSpec((_ROWS, _W), lambda b, r: (b * rblocks + r, 0)),
            pl.BlockSpec(memory_space=pltpu.SMEM),
        ],
        out_shape=[
            jax.ShapeDtypeStruct((_QROWS, _W), jnp.float32),
            jax.ShapeDtypeStruct((1, 1), jnp.float32),
        ],
    )(output, target)


def _hist_body(nll_hbm, cnt_out, sum_out, cnt_tab, sum_tab, cnt_m, sum_m,
               buf0, buf1, sem0, sem1):
    c = lax.axis_index("c")
    s = lax.axis_index("s")
    wid = s * 2 + c
    lanes = lax.iota(jnp.int32, 16)
    ones = jnp.full((16,), 1.0, jnp.float32)
    zeros = jnp.zeros((16,), jnp.float32)
    sublanes = [lanes + su * _NSB for su in range(_SUB)]

    def _zero(i, carry):
        for u in range(4):
            cnt_tab[pl.ds((i * 4 + u) * 16, 16)] = zeros
            sum_tab[pl.ds((i * 4 + u) * 16, 16)] = zeros
        return carry

    lax.fori_loop(0, (_SUB * _NSB) // (16 * 4), _zero, 0)

    base_row = wid * _TROWS
    npairs = _TROWS // (2 * _CROWS)

    def _rows(buf, r, carry):
        for j in range(_W // 16):
            v = buf[r, pl.ds(j * 16, 16)]
            b = jnp.clip(v * _INV_W, 0.0, float(_NB - 1))
            idx = b.astype(jnp.int32) * 16 + sublanes[j % _SUB]
            plsc.addupdate_scatter(cnt_tab, [idx], ones)
            plsc.addupdate_scatter(sum_tab, [idx], v)
        return carry

    def _start(g, buf, sem):
        return pltpu.async_copy(
            nll_hbm.at[pl.ds(base_row + g * _CROWS, _CROWS)], buf, sem
        )

    def _wait(g, buf, sem):
        pltpu.make_async_copy(
            nll_hbm.at[pl.ds(base_row + g * _CROWS, _CROWS)], buf, sem
        ).wait()

    _start(0, buf0, sem0)

    def _pair(h, carry):
        g0 = h * 2
        _start(g0 + 1, buf1, sem1)
        _wait(g0, buf0, sem0)
        lax.fori_loop(0, _CROWS, lambda r, cc: _rows(buf0, r, cc), carry)

        @pl.when(h < npairs - 1)
        def _():
            _start(g0 + 2, buf0, sem0)

        _wait(g0 + 1, buf1, sem1)
        lax.fori_loop(0, _CROWS, lambda r, cc: _rows(buf1, r, cc), carry)
        return carry

    lax.fori_loop(0, npairs, _pair, 0)

    if _SUB == 1:
        pltpu.sync_copy(cnt_tab, cnt_out.at[wid])
        pltpu.sync_copy(sum_tab, sum_out.at[wid])
    else:
        def _fold(w, carry):
            ca = cnt_tab[pl.ds(w * 16, 16)]
            sa = sum_tab[pl.ds(w * 16, 16)]
            for su in range(1, _SUB):
                ca = ca + cnt_tab[pl.ds(su * _NSB + w * 16, 16)]
                sa = sa + sum_tab[pl.ds(su * _NSB + w * 16, 16)]
            cnt_m[pl.ds(w * 16, 16)] = ca
            sum_m[pl.ds(w * 16, 16)] = sa
            return carry

        lax.fori_loop(0, _NSB // 16, _fold, 0)

        pltpu.sync_copy(cnt_m, cnt_out.at[wid])
        pltpu.sync_copy(sum_m, sum_out.at[wid])


def _hist_call(nll):
    mesh = plsc.VectorSubcoreMesh(core_axis_name="c", subcore_axis_name="s")
    fn = pl.kernel(
        _hist_body,
        out_type=(
            jax.ShapeDtypeStruct((_NW, _NSB), jnp.float32),
            jax.ShapeDtypeStruct((_NW, _NSB), jnp.float32),
        ),
        mesh=mesh,
        compiler_params=pltpu.CompilerParams(needs_layout_passes=False),
        scratch_types=[
            pltpu.VMEM((_SUB * _NSB,), jnp.float32),
            pltpu.VMEM((_SUB * _NSB,), jnp.float32),
            pltpu.VMEM((_NSB,), jnp.float32),
            pltpu.VMEM((_NSB,), jnp.float32),
            pltpu.VMEM((_CROWS, _W), jnp.float32),
            pltpu.VMEM((_CROWS, _W), jnp.float32),
            pltpu.SemaphoreType.DMA,
            pltpu.SemaphoreType.DMA,
        ],
    )
    return fn(nll)


def _suffix_incl(x):
    n = x.shape[1]
    sft = 1
    while sft < n:
        x = x + jnp.concatenate(
            [x[:, sft:], jnp.zeros((1, sft), jnp.float32)], axis=1
        )
        sft *= 2
    return x


def _group_suffix(x, grp, width):
    sft = 1
    while sft < width:
        sh = jnp.concatenate(
            [x[:, sft:], jnp.zeros((1, sft), jnp.float32)], axis=1
        )
        x = x + jnp.where(grp < width - sft, sh, 0.0)
        sft *= 2
    return x


def _sel_body(*refs):
    cnt_refs = refs[:_NQ]
    sum_refs = refs[_NQ : 2 * _NQ]
    tot_ref = refs[2 * _NQ]
    topk_ref, raw_ref = refs[2 * _NQ + 1], refs[2 * _NQ + 2]
    cnt = jnp.zeros((1, _NSB), jnp.float32)
    sm = jnp.zeros((1, _NSB), jnp.float32)
    for q in range(_NQ):
        cnt = cnt + jnp.sum(cnt_refs[q][...], axis=0, keepdims=True)
        sm = sm + jnp.sum(sum_refs[q][...], axis=0, keepdims=True)
    pos = lax.broadcasted_iota(jnp.int32, (1, _NSB), 1)
    grp = pos % 16
    base = (grp == 0).astype(jnp.float32)
    counts = _group_suffix(cnt, grp, 16) * base
    sums = _group_suffix(sm, grp, 16) * base
    rc = _suffix_incl(counts)
    rs = _suffix_incl(sums)
    above_c = rc - counts
    above_s = rs - sums
    kf = jnp.float32(_K)
    hit = ((above_c < kf) & (above_c + counts >= kf)).astype(jnp.float32)
    hit = hit * base
    center = ((pos // 16).astype(jnp.float32) + 0.5) * (1.0 / _INV_W)
    a_sel = jnp.sum(above_c * hit)
    s_sel = jnp.sum(above_s * hit)
    t_sel = jnp.sum(center * hit)
    topk_sum = s_sel + (kf - a_sel) * t_sel
    topk_ref[0, 0] = topk_sum / kf
    raw_ref[0, 0] = tot_ref[0, 0] / jnp.float32(_NPIX)


def _sel_call(cnts, sms, tot):
    return pl.pallas_call(
        _sel_body,
        in_specs=(
            [pl.BlockSpec(memory_space=pltpu.VMEM) for _ in range(2 * _NQ)]
            + [pl.BlockSpec(memory_space=pltpu.SMEM)]
        ),
        out_specs=[
            pl.BlockSpec(memory_space=pltpu.SMEM),
            pl.BlockSpec(memory_space=pltpu.SMEM),
        ],
        out_shape=[
            jax.ShapeDtypeStruct((1, 1), jnp.float32),
            jax.ShapeDtypeStruct((1, 1), jnp.float32),
        ],
    )(*cnts, *sms, tot)


def kernel(output, target, it):
    cnts, sms, tots = [], [], []
    for q in range(_NQ):
        nll_q, tot_q = _ce_call(output, target, q)
        cnt_q, sm_q = _hist_call(nll_q)
        cnts.append(cnt_q)
        sms.append(sm_q)
        tots.append(tot_q)
    tot = sum(tots[1:], tots[0])
    topk, raw = _sel_call(cnts, sms, tot)
    topk_mean = topk[0, 0]
    raw_mean = raw[0, 0]

    it_arr = jnp.asarray(it)
    itf = it_arr.astype(jnp.float32)
    ramp = jnp.float32(_TOP_P) + jnp.float32(1.0 - _TOP_P) * (
        (jnp.float32(_END_WARM) - itf) / jnp.float32(_END_WARM - _START_WARM)
    )
    this_p = jnp.where(
        it_arr < _START_WARM,
        jnp.float32(1.0),
        jnp.where(it_arr > _END_WARM, jnp.float32(_TOP_P), ramp),
    )
    loss = jnp.where(it_arr < _START_WARM, raw_mean, topk_mean)
    return (loss, this_p, raw_mean)
